# Initial kernel scaffold; baseline (speedup 1.0000x reference)
import jax
import jax.numpy as jnp
from jax.experimental import pallas as pl


def kernel(r, R, w_init, nuc_table, layers_Ww, layers_Wh, layers_Wu, atom_types):
    raise NotImplementedError("write your pallas kernel here")


# speedup vs baseline: 29.4334x; 29.4334x over previous
import jax
import jax.numpy as jnp
from jax.experimental import pallas as pl
from jax.experimental.pallas import tpu as pltpu

N_E = 512
N_HALF = 256
N_N = 32
D_ = 128
P_ = 64
L_ = 3
_T8 = N_HALF // 8

_HIGH = jax.lax.Precision.HIGHEST
_DEF = jax.lax.Precision.DEFAULT


def _b16(x):
    return x.astype(jnp.bfloat16).astype(jnp.float32)


def _edge_feat_planes(dx, dy, dz):
    d2 = dx * dx + dy * dy + dz * dz
    d = jnp.sqrt(d2)
    u = jnp.exp(d2 * (-0.25))
    u2 = u * u
    u4 = u2 * u2
    u8 = u4 * u4
    return (d, dx, dy, dz, u8, u4, u2, u)


def _init_kernel(rT_ref, R_ref, w_initT_ref, nuc_tableT_ref, atom_ref,
                 featneT_ref, nuc_embT_ref, h0T_ref):
    rT = rT_ref[...]
    R = R_ref[...]

    for a in range(N_N):
        dx = rT[0:1, :] - R[a:a + 1, 0:1]
        dy = rT[1:2, :] - R[a:a + 1, 1:2]
        dz = rT[2:3, :] - R[a:a + 1, 2:3]
        rows = jnp.concatenate(_edge_feat_planes(dx, dy, dz), axis=0)
        featneT_ref[a * 8:(a + 1) * 8, :] = rows

    t_iota = jax.lax.broadcasted_iota(jnp.int32, (4, N_N), 0)
    onehot = (t_iota == atom_ref[...]).astype(jnp.float32)
    nuc_embT_ref[...] = jax.lax.dot_general(
        nuc_tableT_ref[...], onehot, (((1,), (0,)), ((), ())),
        precision=_HIGH)

    h0T_ref[...] = jnp.tanh(jax.lax.dot_general(
        w_initT_ref[...], featneT_ref[...], (((1,), (0,)), ((), ())),
        precision=_DEF))


def _pair_accumulate(acc, i_base, Ww, hs3_ref, rT_j, r_ref):
    Wwb = _b16(Ww)
    wcols = [Wwb[:, k:k + 1] for k in range(8)]

    def tile_body(t, acc):
        ri = r_ref[pl.ds(i_base + t * 8, 8), :]
        dx = rT_j[0:1, :] - ri[:, 0:1]
        dy = rT_j[1:2, :] - ri[:, 1:2]
        dz = rT_j[2:3, :] - ri[:, 2:3]
        feats = [_b16(f) for f in _edge_feat_planes(dx, dy, dz)]
        hs_tile = hs3_ref[pl.ds(t, 1), :, :]
        for ti in range(8):
            wp = feats[0][ti:ti + 1, :] * wcols[0]
            for k in range(1, 8):
                wp = wp + feats[k][ti:ti + 1, :] * wcols[k]
            acc = acc + jnp.tanh(wp) * hs_tile[0, :, ti:ti + 1]
        return acc

    return jax.lax.fori_loop(0, _T8, tile_body, acc)


def _layer_kernel(rT_j_ref, r_ref, hT_ref, featneT_j_ref, nuc_embT_ref,
                  WwT_ref, WhT_ref, WuT_ref, houtT_ref,
                  hs3_up_ref, hs3_dn_ref):
    jh = pl.program_id(0)
    is_up = (jh == 0)

    hT = hT_ref[...]
    dn = (((1,), (0,)), ((), ()))
    hs_same = jax.lax.dot_general(WhT_ref[1], hT, dn, precision=_DEF)
    hs_anti = jax.lax.dot_general(WhT_ref[2], hT, dn, precision=_DEF)
    hs_ne = jax.lax.dot_general(WhT_ref[0], nuc_embT_ref[...], dn,
                                precision=_DEF)

    Ww_same, Ww_anti = WwT_ref[1], WwT_ref[2]
    Ww_up = jnp.where(is_up, Ww_same, Ww_anti)
    Ww_dn = jnp.where(is_up, Ww_anti, Ww_same)
    hs_up = jnp.where(is_up, hs_same[:, :N_HALF], hs_anti[:, :N_HALF])
    hs_dn = jnp.where(is_up, hs_anti[:, N_HALF:], hs_same[:, N_HALF:])
    for t in range(_T8):
        hs3_up_ref[t, :, :] = hs_up[:, t * 8:(t + 1) * 8]
        hs3_dn_ref[t, :, :] = hs_dn[:, t * 8:(t + 1) * 8]

    rT_j = rT_j_ref[...]
    zeros = jnp.zeros((P_, N_HALF), jnp.float32)

    acc_up = _pair_accumulate(zeros, 0, Ww_up, hs3_up_ref, rT_j, r_ref)
    acc_dn = _pair_accumulate(zeros, N_HALF, Ww_dn, hs3_dn_ref, rT_j, r_ref)

    acc_same = jnp.where(is_up, acc_up, acc_dn)
    acc_anti = jnp.where(is_up, acc_dn, acc_up)

    Wb = _b16(Ww_same)
    wdiag = jnp.tanh(((Wb[:, 4:5] + Wb[:, 5:6]) + Wb[:, 6:7]) + Wb[:, 7:8])
    hs_same_j = jnp.where(is_up, hs_same[:, :N_HALF], hs_same[:, N_HALF:])
    acc_same = acc_same - wdiag * hs_same_j

    WwNeb = _b16(WwT_ref[0])
    wcols_ne = [WwNeb[:, k:k + 1] for k in range(8)]
    acc_ne = zeros
    for a in range(N_N):
        wp = _b16(featneT_j_ref[a * 8:a * 8 + 1, :]) * wcols_ne[0]
        for k in range(1, 8):
            wp = wp + _b16(featneT_j_ref[a * 8 + k:a * 8 + k + 1, :]) * wcols_ne[k]
        acc_ne = acc_ne + jnp.tanh(wp) * hs_ne[:, a:a + 1]

    mcat = jnp.concatenate([acc_ne, acc_same, acc_anti], axis=0)
    hT_j = jnp.where(is_up, hT[:, :N_HALF], hT[:, N_HALF:])
    houtT_ref[...] = hT_j + jnp.tanh(jax.lax.dot_general(
        WuT_ref[...], mcat, dn, precision=_DEF))


@jax.jit
def kernel(r, R, w_init, nuc_table, layers_Ww, layers_Wh, layers_Wu,
           atom_types):
    rT = r.T
    w_initT = w_init.T
    nuc_tableT = nuc_table.T
    WwT = jnp.swapaxes(layers_Ww, 2, 3)
    WhT = jnp.swapaxes(layers_Wh, 2, 3)
    WuT = jnp.swapaxes(layers_Wu, 1, 2)
    atom2d = atom_types.reshape(1, N_N)

    featneT, nuc_embT, hT = pl.pallas_call(
        _init_kernel,
        out_shape=(
            jax.ShapeDtypeStruct((N_N * 8, N_E), jnp.float32),
            jax.ShapeDtypeStruct((D_, N_N), jnp.float32),
            jax.ShapeDtypeStruct((D_, N_E), jnp.float32),
        ),
    )(rT, R, w_initT, nuc_tableT, atom2d)

    grid = (2,)
    layer_call = pl.pallas_call(
        _layer_kernel,
        grid=grid,
        in_specs=[
            pl.BlockSpec((3, N_HALF), lambda j: (0, j)),
            pl.BlockSpec((N_E, 3), lambda j: (0, 0)),
            pl.BlockSpec((D_, N_E), lambda j: (0, 0)),
            pl.BlockSpec((N_N * 8, N_HALF), lambda j: (0, j)),
            pl.BlockSpec((D_, N_N), lambda j: (0, 0)),
            pl.BlockSpec((3, P_, 8), lambda j: (0, 0, 0)),
            pl.BlockSpec((3, P_, D_), lambda j: (0, 0, 0)),
            pl.BlockSpec((D_, 3 * P_), lambda j: (0, 0)),
        ],
        out_specs=pl.BlockSpec((D_, N_HALF), lambda j: (0, j)),
        out_shape=jax.ShapeDtypeStruct((D_, N_E), jnp.float32),
        scratch_shapes=[
            pltpu.VMEM((_T8, P_, 8), jnp.float32),
            pltpu.VMEM((_T8, P_, 8), jnp.float32),
        ],
        compiler_params=pltpu.CompilerParams(
            dimension_semantics=("arbitrary",)),
    )

    for l in range(L_):
        hT = layer_call(rT, r, hT, featneT, nuc_embT,
                        WwT[l], WhT[l], WuT[l])
    return hT.T


# speedup vs baseline: 29.5590x; 1.0043x over previous
import jax
import jax.numpy as jnp
from jax.experimental import pallas as pl
from jax.experimental.pallas import tpu as pltpu

N_E = 512
N_HALF = 256
N_N = 32
D_ = 128
P_ = 64
L_ = 3
_T8 = N_HALF // 8

_HIGH = jax.lax.Precision.HIGHEST
_DEF = jax.lax.Precision.DEFAULT


def _b16(x):
    return x.astype(jnp.bfloat16).astype(jnp.float32)


def _edge_feat_planes(dx, dy, dz):
    d2 = dx * dx + dy * dy + dz * dz
    d = jnp.sqrt(d2)
    u = jnp.exp(d2 * (-0.25))
    u2 = u * u
    u4 = u2 * u2
    u8 = u4 * u4
    return (d, dx, dy, dz, u8, u4, u2, u)


def _init_kernel(rT_ref, R_ref, w_initT_ref, nuc_tableT_ref, atom_ref,
                 featneT_ref, nuc_embT_ref, h0T_ref):
    rT = rT_ref[...]
    R = R_ref[...]

    for a in range(N_N):
        dx = rT[0:1, :] - R[a:a + 1, 0:1]
        dy = rT[1:2, :] - R[a:a + 1, 1:2]
        dz = rT[2:3, :] - R[a:a + 1, 2:3]
        rows = jnp.concatenate(_edge_feat_planes(dx, dy, dz), axis=0)
        featneT_ref[a * 8:(a + 1) * 8, :] = rows

    t_iota = jax.lax.broadcasted_iota(jnp.int32, (4, N_N), 0)
    onehot = (t_iota == atom_ref[...]).astype(jnp.float32)
    nuc_embT_ref[...] = jax.lax.dot_general(
        nuc_tableT_ref[...], onehot, (((1,), (0,)), ((), ())),
        precision=_HIGH)

    h0T_ref[...] = jnp.tanh(jax.lax.dot_general(
        w_initT_ref[...], featneT_ref[...], (((1,), (0,)), ((), ())),
        precision=_DEF))


def _pair_accumulate(acc, i_base, Ww, hs3_ref, rT_j, r_ref):
    Wwb = _b16(Ww)
    wcols = [Wwb[:, k:k + 1] for k in range(8)]

    def tile_body(t, acc):
        ri = r_ref[pl.ds(i_base + t * 8, 8), :]
        dx = rT_j[0:1, :] - ri[:, 0:1]
        dy = rT_j[1:2, :] - ri[:, 1:2]
        dz = rT_j[2:3, :] - ri[:, 2:3]
        feats = [_b16(f) for f in _edge_feat_planes(dx, dy, dz)]
        hs_tile = hs3_ref[pl.ds(t, 1), :, :]
        for ti in range(8):
            wp = feats[0][ti:ti + 1, :] * wcols[0]
            for k in range(1, 8):
                wp = wp + feats[k][ti:ti + 1, :] * wcols[k]
            acc = acc + jnp.tanh(wp) * hs_tile[0, :, ti:ti + 1]
        return acc

    return jax.lax.fori_loop(0, _T8, tile_body, acc)


def _layer_kernel(rT_j_ref, r_ref, hT_ref, featneT_j_ref, nuc_embT_ref,
                  WwT_ref, WhT_ref, WuT_ref, houtT_ref,
                  hs3_up_ref, hs3_dn_ref):
    jh = pl.program_id(0)
    is_up = (jh == 0)

    hT = hT_ref[...]
    dn = (((1,), (0,)), ((), ()))
    hs_same = jax.lax.dot_general(WhT_ref[1], hT, dn, precision=_DEF)
    hs_anti = jax.lax.dot_general(WhT_ref[2], hT, dn, precision=_DEF)
    hs_ne = jax.lax.dot_general(WhT_ref[0], nuc_embT_ref[...], dn,
                                precision=_DEF)

    Ww_same, Ww_anti = WwT_ref[1], WwT_ref[2]
    Ww_up = jnp.where(is_up, Ww_same, Ww_anti)
    Ww_dn = jnp.where(is_up, Ww_anti, Ww_same)
    hs_up = jnp.where(is_up, hs_same[:, :N_HALF], hs_anti[:, :N_HALF])
    hs_dn = jnp.where(is_up, hs_anti[:, N_HALF:], hs_same[:, N_HALF:])
    for t in range(_T8):
        hs3_up_ref[t, :, :] = hs_up[:, t * 8:(t + 1) * 8]
        hs3_dn_ref[t, :, :] = hs_dn[:, t * 8:(t + 1) * 8]

    rT_j = rT_j_ref[...]
    zeros = jnp.zeros((P_, N_HALF), jnp.float32)

    acc_up = _pair_accumulate(zeros, 0, Ww_up, hs3_up_ref, rT_j, r_ref)
    acc_dn = _pair_accumulate(zeros, N_HALF, Ww_dn, hs3_dn_ref, rT_j, r_ref)

    acc_same = jnp.where(is_up, acc_up, acc_dn)
    acc_anti = jnp.where(is_up, acc_dn, acc_up)

    Wb = _b16(Ww_same)
    wdiag = jnp.tanh(((Wb[:, 4:5] + Wb[:, 5:6]) + Wb[:, 6:7]) + Wb[:, 7:8])
    hs_same_j = jnp.where(is_up, hs_same[:, :N_HALF], hs_same[:, N_HALF:])
    acc_same = acc_same - wdiag * hs_same_j

    WwNeb = _b16(WwT_ref[0])
    wcols_ne = [WwNeb[:, k:k + 1] for k in range(8)]
    acc_ne = zeros
    for a in range(N_N):
        wp = _b16(featneT_j_ref[a * 8:a * 8 + 1, :]) * wcols_ne[0]
        for k in range(1, 8):
            wp = wp + _b16(featneT_j_ref[a * 8 + k:a * 8 + k + 1, :]) * wcols_ne[k]
        acc_ne = acc_ne + jnp.tanh(wp) * hs_ne[:, a:a + 1]

    mcat = jnp.concatenate([acc_ne, acc_same, acc_anti], axis=0)
    hT_j = jnp.where(is_up, hT[:, :N_HALF], hT[:, N_HALF:])
    houtT_ref[...] = hT_j + jnp.tanh(jax.lax.dot_general(
        WuT_ref[...], mcat, dn, precision=_DEF))


@jax.jit
def kernel(r, R, w_init, nuc_table, layers_Ww, layers_Wh, layers_Wu,
           atom_types):
    rT = r.T
    w_initT = w_init.T
    nuc_tableT = nuc_table.T
    WwT = jnp.swapaxes(layers_Ww, 2, 3)
    WhT = jnp.swapaxes(layers_Wh, 2, 3)
    WuT = jnp.swapaxes(layers_Wu, 1, 2)
    atom2d = atom_types.reshape(1, N_N)

    featneT, nuc_embT, hT = pl.pallas_call(
        _init_kernel,
        out_shape=(
            jax.ShapeDtypeStruct((N_N * 8, N_E), jnp.float32),
            jax.ShapeDtypeStruct((D_, N_N), jnp.float32),
            jax.ShapeDtypeStruct((D_, N_E), jnp.float32),
        ),
    )(rT, R, w_initT, nuc_tableT, atom2d)

    grid = (2,)
    layer_call = pl.pallas_call(
        _layer_kernel,
        grid=grid,
        in_specs=[
            pl.BlockSpec((3, N_HALF), lambda j: (0, j)),
            pl.BlockSpec((N_E, 3), lambda j: (0, 0)),
            pl.BlockSpec((D_, N_E), lambda j: (0, 0)),
            pl.BlockSpec((N_N * 8, N_HALF), lambda j: (0, j)),
            pl.BlockSpec((D_, N_N), lambda j: (0, 0)),
            pl.BlockSpec((3, P_, 8), lambda j: (0, 0, 0)),
            pl.BlockSpec((3, P_, D_), lambda j: (0, 0, 0)),
            pl.BlockSpec((D_, 3 * P_), lambda j: (0, 0)),
        ],
        out_specs=pl.BlockSpec((D_, N_HALF), lambda j: (0, j)),
        out_shape=jax.ShapeDtypeStruct((D_, N_E), jnp.float32),
        scratch_shapes=[
            pltpu.VMEM((_T8, P_, 8), jnp.float32),
            pltpu.VMEM((_T8, P_, 8), jnp.float32),
        ],
        compiler_params=pltpu.CompilerParams(
            dimension_semantics=("parallel",)),
    )

    for l in range(L_):
        hT = layer_call(rT, r, hT, featneT, nuc_embT,
                        WwT[l], WhT[l], WuT[l])
    return hT.T


# speedup vs baseline: 40.4463x; 1.3683x over previous
import jax
import jax.numpy as jnp
from jax.experimental import pallas as pl
from jax.experimental.pallas import tpu as pltpu

N_E = 512
N_HALF = 256
N_N = 32
D_ = 128
P_ = 64
L_ = 3
_T8 = N_HALF // 8

_HIGH = jax.lax.Precision.HIGHEST
_DEF = jax.lax.Precision.DEFAULT


def _b16(x):
    return x.astype(jnp.bfloat16).astype(jnp.float32)


def _edge_feat_planes(dx, dy, dz):
    d2 = dx * dx + dy * dy + dz * dz
    d = jnp.sqrt(d2)
    u = jnp.exp(d2 * (-0.25))
    u2 = u * u
    u4 = u2 * u2
    u8 = u4 * u4
    return (d, dx, dy, dz, u8, u4, u2, u)


def _init_kernel(rT_ref, R_ref, w_initT_ref, nuc_tableT_ref, atom_ref,
                 featneT_ref, nuc_embT_ref, h0T_ref):
    rT = rT_ref[...]
    R = R_ref[...]

    for a in range(N_N):
        dx = rT[0:1, :] - R[a:a + 1, 0:1]
        dy = rT[1:2, :] - R[a:a + 1, 1:2]
        dz = rT[2:3, :] - R[a:a + 1, 2:3]
        rows = jnp.concatenate(_edge_feat_planes(dx, dy, dz), axis=0)
        featneT_ref[a * 8:(a + 1) * 8, :] = rows

    t_iota = jax.lax.broadcasted_iota(jnp.int32, (4, N_N), 0)
    onehot = (t_iota == atom_ref[...]).astype(jnp.float32)
    nuc_embT_ref[...] = jax.lax.dot_general(
        nuc_tableT_ref[...], onehot, (((1,), (0,)), ((), ())),
        precision=_HIGH)

    h0T_ref[...] = jnp.tanh(jax.lax.dot_general(
        w_initT_ref[...], featneT_ref[...], (((1,), (0,)), ((), ())),
        precision=_DEF))


def _pair_accumulate(acc, i_base, Ww, hs3_ref, rT_j, r_ref):
    dn = (((1,), (0,)), ((), ()))

    def tile_body(t, acc):
        ri = r_ref[pl.ds(i_base + t * 8, 8), :]
        dx = rT_j[0:1, :] - ri[:, 0:1]
        dy = rT_j[1:2, :] - ri[:, 1:2]
        dz = rT_j[2:3, :] - ri[:, 2:3]
        feats = _edge_feat_planes(dx, dy, dz)
        hs_tile = hs3_ref[pl.ds(t, 1), :, :]
        for ti in range(8):
            F = jnp.concatenate([f[ti:ti + 1, :] for f in feats], axis=0)
            wp = jax.lax.dot_general(Ww, F, dn, precision=_DEF)
            acc = acc + jnp.tanh(wp) * hs_tile[0, :, ti:ti + 1]
        return acc

    return jax.lax.fori_loop(0, _T8, tile_body, acc)


def _layer_kernel(rT_j_ref, r_ref, hT_ref, featneT_j_ref, nuc_embT_ref,
                  WwT_ref, WhT_ref, WuT_ref, houtT_ref,
                  hs3_up_ref, hs3_dn_ref):
    jh = pl.program_id(0)
    is_up = (jh == 0)

    hT = hT_ref[...]
    dn = (((1,), (0,)), ((), ()))
    hs_same = jax.lax.dot_general(WhT_ref[1], hT, dn, precision=_DEF)
    hs_anti = jax.lax.dot_general(WhT_ref[2], hT, dn, precision=_DEF)
    hs_ne = jax.lax.dot_general(WhT_ref[0], nuc_embT_ref[...], dn,
                                precision=_DEF)

    Ww_same, Ww_anti = WwT_ref[1], WwT_ref[2]
    Ww_up = jnp.where(is_up, Ww_same, Ww_anti)
    Ww_dn = jnp.where(is_up, Ww_anti, Ww_same)
    hs_up = jnp.where(is_up, hs_same[:, :N_HALF], hs_anti[:, :N_HALF])
    hs_dn = jnp.where(is_up, hs_anti[:, N_HALF:], hs_same[:, N_HALF:])
    for t in range(_T8):
        hs3_up_ref[t, :, :] = hs_up[:, t * 8:(t + 1) * 8]
        hs3_dn_ref[t, :, :] = hs_dn[:, t * 8:(t + 1) * 8]

    rT_j = rT_j_ref[...]
    zeros = jnp.zeros((P_, N_HALF), jnp.float32)

    acc_up = _pair_accumulate(zeros, 0, Ww_up, hs3_up_ref, rT_j, r_ref)
    acc_dn = _pair_accumulate(zeros, N_HALF, Ww_dn, hs3_dn_ref, rT_j, r_ref)

    acc_same = jnp.where(is_up, acc_up, acc_dn)
    acc_anti = jnp.where(is_up, acc_dn, acc_up)

    Wb = _b16(Ww_same)
    wdiag = jnp.tanh(((Wb[:, 4:5] + Wb[:, 5:6]) + Wb[:, 6:7]) + Wb[:, 7:8])
    hs_same_j = jnp.where(is_up, hs_same[:, :N_HALF], hs_same[:, N_HALF:])
    acc_same = acc_same - wdiag * hs_same_j

    WwNe = WwT_ref[0]
    acc_ne = zeros
    for a in range(N_N):
        F = featneT_j_ref[a * 8:(a + 1) * 8, :]
        wp = jax.lax.dot_general(WwNe, F, dn, precision=_DEF)
        acc_ne = acc_ne + jnp.tanh(wp) * hs_ne[:, a:a + 1]

    mcat = jnp.concatenate([acc_ne, acc_same, acc_anti], axis=0)
    hT_j = jnp.where(is_up, hT[:, :N_HALF], hT[:, N_HALF:])
    houtT_ref[...] = hT_j + jnp.tanh(jax.lax.dot_general(
        WuT_ref[...], mcat, dn, precision=_DEF))


@jax.jit
def kernel(r, R, w_init, nuc_table, layers_Ww, layers_Wh, layers_Wu,
           atom_types):
    rT = r.T
    w_initT = w_init.T
    nuc_tableT = nuc_table.T
    WwT = jnp.swapaxes(layers_Ww, 2, 3)
    WhT = jnp.swapaxes(layers_Wh, 2, 3)
    WuT = jnp.swapaxes(layers_Wu, 1, 2)
    atom2d = atom_types.reshape(1, N_N)

    featneT, nuc_embT, hT = pl.pallas_call(
        _init_kernel,
        out_shape=(
            jax.ShapeDtypeStruct((N_N * 8, N_E), jnp.float32),
            jax.ShapeDtypeStruct((D_, N_N), jnp.float32),
            jax.ShapeDtypeStruct((D_, N_E), jnp.float32),
        ),
    )(rT, R, w_initT, nuc_tableT, atom2d)

    grid = (2,)
    layer_call = pl.pallas_call(
        _layer_kernel,
        grid=grid,
        in_specs=[
            pl.BlockSpec((3, N_HALF), lambda j: (0, j)),
            pl.BlockSpec((N_E, 3), lambda j: (0, 0)),
            pl.BlockSpec((D_, N_E), lambda j: (0, 0)),
            pl.BlockSpec((N_N * 8, N_HALF), lambda j: (0, j)),
            pl.BlockSpec((D_, N_N), lambda j: (0, 0)),
            pl.BlockSpec((3, P_, 8), lambda j: (0, 0, 0)),
            pl.BlockSpec((3, P_, D_), lambda j: (0, 0, 0)),
            pl.BlockSpec((D_, 3 * P_), lambda j: (0, 0)),
        ],
        out_specs=pl.BlockSpec((D_, N_HALF), lambda j: (0, j)),
        out_shape=jax.ShapeDtypeStruct((D_, N_E), jnp.float32),
        scratch_shapes=[
            pltpu.VMEM((_T8, P_, 8), jnp.float32),
            pltpu.VMEM((_T8, P_, 8), jnp.float32),
        ],
        compiler_params=pltpu.CompilerParams(
            dimension_semantics=("parallel",)),
    )

    for l in range(L_):
        hT = layer_call(rT, r, hT, featneT, nuc_embT,
                        WwT[l], WhT[l], WuT[l])
    return hT.T


# speedup vs baseline: 56.5963x; 1.3993x over previous
import jax
import jax.numpy as jnp
from jax.experimental import pallas as pl
from jax.experimental.pallas import tpu as pltpu

N_E = 512
N_HALF = 256
N_N = 32
D_ = 128
P_ = 64
L_ = 3
_T8 = N_HALF // 8

_HIGH = jax.lax.Precision.HIGHEST
_DEF = jax.lax.Precision.DEFAULT


def _b16(x):
    return x.astype(jnp.bfloat16).astype(jnp.float32)


def _edge_feat_planes(dx, dy, dz):
    d2 = dx * dx + dy * dy + dz * dz
    d = jnp.sqrt(d2)
    u = jnp.exp(d2 * (-0.25))
    u2 = u * u
    u4 = u2 * u2
    u8 = u4 * u4
    return (d, dx, dy, dz, u8, u4, u2, u)


def _init_kernel(rT_ref, R_ref, w_initT_ref, nuc_tableT_ref, atom_ref,
                 featneT_ref, nuc_embT_ref, h0T_ref):
    rT = rT_ref[...]
    R = R_ref[...]

    for a in range(N_N):
        dx = rT[0:1, :] - R[a:a + 1, 0:1]
        dy = rT[1:2, :] - R[a:a + 1, 1:2]
        dz = rT[2:3, :] - R[a:a + 1, 2:3]
        rows = jnp.concatenate(_edge_feat_planes(dx, dy, dz), axis=0)
        featneT_ref[a * 8:(a + 1) * 8, :] = rows

    t_iota = jax.lax.broadcasted_iota(jnp.int32, (4, N_N), 0)
    onehot = (t_iota == atom_ref[...]).astype(jnp.float32)
    nuc_embT_ref[...] = jax.lax.dot_general(
        nuc_tableT_ref[...], onehot, (((1,), (0,)), ((), ())),
        precision=_HIGH)

    h0T_ref[...] = jnp.tanh(jax.lax.dot_general(
        w_initT_ref[...], featneT_ref[...], (((1,), (0,)), ((), ())),
        precision=_DEF))


def _pair_accumulate(acc, i_base, Ww, hs3_ref, rT_j, r_ref):
    dn = (((1,), (0,)), ((), ()))

    def tile_body(t, acc):
        ri = r_ref[pl.ds(i_base + t * 8, 8), :]
        dx = rT_j[0:1, :] - ri[:, 0:1]
        dy = rT_j[1:2, :] - ri[:, 1:2]
        dz = rT_j[2:3, :] - ri[:, 2:3]
        feats = _edge_feat_planes(dx, dy, dz)
        hs_tile = hs3_ref[pl.ds(t, 1), :, :]
        Fs = [jnp.concatenate([f[ti:ti + 1, :] for f in feats], axis=0)
              for ti in range(8)]
        wps = [jax.lax.dot_general(Ww, F, dn, precision=_DEF) for F in Fs]
        for ti in range(8):
            acc = acc + jnp.tanh(wps[ti]) * hs_tile[0, :, ti:ti + 1]
        return acc

    return jax.lax.fori_loop(0, _T8, tile_body, acc, unroll=2)


def _layer_kernel(rT_j_ref, r_ref, hT_ref, featneT_j_ref, nuc_embT_ref,
                  WwT_ref, WhT_ref, WuT_ref, houtT_ref,
                  hs3_up_ref, hs3_dn_ref):
    jh = pl.program_id(0)
    is_up = (jh == 0)

    hT = hT_ref[...]
    dn = (((1,), (0,)), ((), ()))
    hs_same = jax.lax.dot_general(WhT_ref[1], hT, dn, precision=_DEF)
    hs_anti = jax.lax.dot_general(WhT_ref[2], hT, dn, precision=_DEF)
    hs_ne = jax.lax.dot_general(WhT_ref[0], nuc_embT_ref[...], dn,
                                precision=_DEF)

    Ww_same, Ww_anti = WwT_ref[1], WwT_ref[2]
    Ww_up = jnp.where(is_up, Ww_same, Ww_anti)
    Ww_dn = jnp.where(is_up, Ww_anti, Ww_same)
    hs_up = jnp.where(is_up, hs_same[:, :N_HALF], hs_anti[:, :N_HALF])
    hs_dn = jnp.where(is_up, hs_anti[:, N_HALF:], hs_same[:, N_HALF:])
    for t in range(_T8):
        hs3_up_ref[t, :, :] = hs_up[:, t * 8:(t + 1) * 8]
        hs3_dn_ref[t, :, :] = hs_dn[:, t * 8:(t + 1) * 8]

    rT_j = rT_j_ref[...]
    zeros = jnp.zeros((P_, N_HALF), jnp.float32)

    acc_up = _pair_accumulate(zeros, 0, Ww_up, hs3_up_ref, rT_j, r_ref)
    acc_dn = _pair_accumulate(zeros, N_HALF, Ww_dn, hs3_dn_ref, rT_j, r_ref)

    acc_same = jnp.where(is_up, acc_up, acc_dn)
    acc_anti = jnp.where(is_up, acc_dn, acc_up)

    Wb = _b16(Ww_same)
    wdiag = jnp.tanh(((Wb[:, 4:5] + Wb[:, 5:6]) + Wb[:, 6:7]) + Wb[:, 7:8])
    hs_same_j = jnp.where(is_up, hs_same[:, :N_HALF], hs_same[:, N_HALF:])
    acc_same = acc_same - wdiag * hs_same_j

    WwNe = WwT_ref[0]
    acc_ne = zeros
    for a in range(N_N):
        F = featneT_j_ref[a * 8:(a + 1) * 8, :]
        wp = jax.lax.dot_general(WwNe, F, dn, precision=_DEF)
        acc_ne = acc_ne + jnp.tanh(wp) * hs_ne[:, a:a + 1]

    mcat = jnp.concatenate([acc_ne, acc_same, acc_anti], axis=0)
    hT_j = jnp.where(is_up, hT[:, :N_HALF], hT[:, N_HALF:])
    houtT_ref[...] = hT_j + jnp.tanh(jax.lax.dot_general(
        WuT_ref[...], mcat, dn, precision=_DEF))


@jax.jit
def kernel(r, R, w_init, nuc_table, layers_Ww, layers_Wh, layers_Wu,
           atom_types):
    rT = r.T
    w_initT = w_init.T
    nuc_tableT = nuc_table.T
    WwT = jnp.swapaxes(layers_Ww, 2, 3)
    WhT = jnp.swapaxes(layers_Wh, 2, 3)
    WuT = jnp.swapaxes(layers_Wu, 1, 2)
    atom2d = atom_types.reshape(1, N_N)

    featneT, nuc_embT, hT = pl.pallas_call(
        _init_kernel,
        out_shape=(
            jax.ShapeDtypeStruct((N_N * 8, N_E), jnp.float32),
            jax.ShapeDtypeStruct((D_, N_N), jnp.float32),
            jax.ShapeDtypeStruct((D_, N_E), jnp.float32),
        ),
    )(rT, R, w_initT, nuc_tableT, atom2d)

    grid = (2,)
    layer_call = pl.pallas_call(
        _layer_kernel,
        grid=grid,
        in_specs=[
            pl.BlockSpec((3, N_HALF), lambda j: (0, j)),
            pl.BlockSpec((N_E, 3), lambda j: (0, 0)),
            pl.BlockSpec((D_, N_E), lambda j: (0, 0)),
            pl.BlockSpec((N_N * 8, N_HALF), lambda j: (0, j)),
            pl.BlockSpec((D_, N_N), lambda j: (0, 0)),
            pl.BlockSpec((3, P_, 8), lambda j: (0, 0, 0)),
            pl.BlockSpec((3, P_, D_), lambda j: (0, 0, 0)),
            pl.BlockSpec((D_, 3 * P_), lambda j: (0, 0)),
        ],
        out_specs=pl.BlockSpec((D_, N_HALF), lambda j: (0, j)),
        out_shape=jax.ShapeDtypeStruct((D_, N_E), jnp.float32),
        scratch_shapes=[
            pltpu.VMEM((_T8, P_, 8), jnp.float32),
            pltpu.VMEM((_T8, P_, 8), jnp.float32),
        ],
        compiler_params=pltpu.CompilerParams(
            dimension_semantics=("parallel",)),
    )

    for l in range(L_):
        hT = layer_call(rT, r, hT, featneT, nuc_embT,
                        WwT[l], WhT[l], WuT[l])
    return hT.T


# speedup vs baseline: 66.5438x; 1.1758x over previous
import jax
import jax.numpy as jnp
from jax.experimental import pallas as pl
from jax.experimental.pallas import tpu as pltpu

N_E = 512
N_HALF = 256
N_N = 32
D_ = 128
P_ = 64
L_ = 3
_T8 = N_HALF // 8

_HIGH = jax.lax.Precision.HIGHEST
_DEF = jax.lax.Precision.DEFAULT


def _b16(x):
    return x.astype(jnp.bfloat16).astype(jnp.float32)


def _edge_feat_planes(dx, dy, dz):
    d2 = dx * dx + dy * dy + dz * dz
    d = jnp.sqrt(d2)
    u = jnp.exp(d2 * (-0.25))
    u2 = u * u
    u4 = u2 * u2
    u8 = u4 * u4
    return (d, dx, dy, dz, u8, u4, u2, u)


def _init_kernel(rT_ref, R_ref, w_initT_ref, nuc_tableT_ref, atom_ref,
                 featneT_ref, nuc_embT_ref, h0T_ref):
    rT = rT_ref[...]
    R = R_ref[...]

    for a in range(N_N):
        dx = rT[0:1, :] - R[a:a + 1, 0:1]
        dy = rT[1:2, :] - R[a:a + 1, 1:2]
        dz = rT[2:3, :] - R[a:a + 1, 2:3]
        rows = jnp.concatenate(_edge_feat_planes(dx, dy, dz), axis=0)
        featneT_ref[a * 8:(a + 1) * 8, :] = rows

    t_iota = jax.lax.broadcasted_iota(jnp.int32, (4, N_N), 0)
    onehot = (t_iota == atom_ref[...]).astype(jnp.float32)
    nuc_embT_ref[...] = jax.lax.dot_general(
        nuc_tableT_ref[...], onehot, (((1,), (0,)), ((), ())),
        precision=_HIGH)

    h0T_ref[...] = jnp.tanh(jax.lax.dot_general(
        w_initT_ref[...], featneT_ref[...], (((1,), (0,)), ((), ())),
        precision=_DEF))


def _pair_accumulate(acc, i_base, Ww, hs3_ref, rT_j, r_ref):
    dn = (((1,), (0,)), ((), ()))

    def tile_body(t, accs):
        acc0, acc1 = accs
        ri = r_ref[pl.ds(i_base + t * 8, 8), :]
        dx = rT_j[0:1, :] - ri[:, 0:1]
        dy = rT_j[1:2, :] - ri[:, 1:2]
        dz = rT_j[2:3, :] - ri[:, 2:3]
        feats = _edge_feat_planes(dx, dy, dz)
        hs_tile = hs3_ref[pl.ds(t, 1), :, :]
        Fs = [jnp.concatenate([f[ti:ti + 1, :] for f in feats], axis=0)
              for ti in range(8)]
        wps = [jax.lax.dot_general(Ww, F, dn, precision=_DEF) for F in Fs]
        for ti in range(0, 8, 2):
            acc0 = acc0 + jnp.tanh(wps[ti]) * hs_tile[0, :, ti:ti + 1]
            acc1 = acc1 + jnp.tanh(wps[ti + 1]) * hs_tile[0, :, ti + 1:ti + 2]
        return acc0, acc1

    acc0, acc1 = jax.lax.fori_loop(
        0, _T8, tile_body, (acc, jnp.zeros_like(acc)), unroll=4)
    return acc0 + acc1


def _layer_kernel(rT_j_ref, r_ref, hT_ref, featneT_j_ref, nuc_embT_ref,
                  WwT_ref, WhT_ref, WuT_ref, houtT_ref,
                  hs3_up_ref, hs3_dn_ref):
    jh = pl.program_id(0)
    is_up = (jh == 0)

    hT = hT_ref[...]
    dn = (((1,), (0,)), ((), ()))
    hs_same = jax.lax.dot_general(WhT_ref[1], hT, dn, precision=_DEF)
    hs_anti = jax.lax.dot_general(WhT_ref[2], hT, dn, precision=_DEF)
    hs_ne = jax.lax.dot_general(WhT_ref[0], nuc_embT_ref[...], dn,
                                precision=_DEF)

    Ww_same, Ww_anti = WwT_ref[1], WwT_ref[2]
    Ww_up = jnp.where(is_up, Ww_same, Ww_anti)
    Ww_dn = jnp.where(is_up, Ww_anti, Ww_same)
    hs_up = jnp.where(is_up, hs_same[:, :N_HALF], hs_anti[:, :N_HALF])
    hs_dn = jnp.where(is_up, hs_anti[:, N_HALF:], hs_same[:, N_HALF:])
    for t in range(_T8):
        hs3_up_ref[t, :, :] = hs_up[:, t * 8:(t + 1) * 8]
        hs3_dn_ref[t, :, :] = hs_dn[:, t * 8:(t + 1) * 8]

    rT_j = rT_j_ref[...]
    zeros = jnp.zeros((P_, N_HALF), jnp.float32)

    acc_up = _pair_accumulate(zeros, 0, Ww_up, hs3_up_ref, rT_j, r_ref)
    acc_dn = _pair_accumulate(zeros, N_HALF, Ww_dn, hs3_dn_ref, rT_j, r_ref)

    acc_same = jnp.where(is_up, acc_up, acc_dn)
    acc_anti = jnp.where(is_up, acc_dn, acc_up)

    Wb = _b16(Ww_same)
    wdiag = jnp.tanh(((Wb[:, 4:5] + Wb[:, 5:6]) + Wb[:, 6:7]) + Wb[:, 7:8])
    hs_same_j = jnp.where(is_up, hs_same[:, :N_HALF], hs_same[:, N_HALF:])
    acc_same = acc_same - wdiag * hs_same_j

    WwNe = WwT_ref[0]
    acc_ne = zeros
    for a in range(N_N):
        F = featneT_j_ref[a * 8:(a + 1) * 8, :]
        wp = jax.lax.dot_general(WwNe, F, dn, precision=_DEF)
        acc_ne = acc_ne + jnp.tanh(wp) * hs_ne[:, a:a + 1]

    mcat = jnp.concatenate([acc_ne, acc_same, acc_anti], axis=0)
    hT_j = jnp.where(is_up, hT[:, :N_HALF], hT[:, N_HALF:])
    houtT_ref[...] = hT_j + jnp.tanh(jax.lax.dot_general(
        WuT_ref[...], mcat, dn, precision=_DEF))


@jax.jit
def kernel(r, R, w_init, nuc_table, layers_Ww, layers_Wh, layers_Wu,
           atom_types):
    rT = r.T
    w_initT = w_init.T
    nuc_tableT = nuc_table.T
    WwT = jnp.swapaxes(layers_Ww, 2, 3)
    WhT = jnp.swapaxes(layers_Wh, 2, 3)
    WuT = jnp.swapaxes(layers_Wu, 1, 2)
    atom2d = atom_types.reshape(1, N_N)

    featneT, nuc_embT, hT = pl.pallas_call(
        _init_kernel,
        out_shape=(
            jax.ShapeDtypeStruct((N_N * 8, N_E), jnp.float32),
            jax.ShapeDtypeStruct((D_, N_N), jnp.float32),
            jax.ShapeDtypeStruct((D_, N_E), jnp.float32),
        ),
    )(rT, R, w_initT, nuc_tableT, atom2d)

    grid = (2,)
    layer_call = pl.pallas_call(
        _layer_kernel,
        grid=grid,
        in_specs=[
            pl.BlockSpec((3, N_HALF), lambda j: (0, j)),
            pl.BlockSpec((N_E, 3), lambda j: (0, 0)),
            pl.BlockSpec((D_, N_E), lambda j: (0, 0)),
            pl.BlockSpec((N_N * 8, N_HALF), lambda j: (0, j)),
            pl.BlockSpec((D_, N_N), lambda j: (0, 0)),
            pl.BlockSpec((3, P_, 8), lambda j: (0, 0, 0)),
            pl.BlockSpec((3, P_, D_), lambda j: (0, 0, 0)),
            pl.BlockSpec((D_, 3 * P_), lambda j: (0, 0)),
        ],
        out_specs=pl.BlockSpec((D_, N_HALF), lambda j: (0, j)),
        out_shape=jax.ShapeDtypeStruct((D_, N_E), jnp.float32),
        scratch_shapes=[
            pltpu.VMEM((_T8, P_, 8), jnp.float32),
            pltpu.VMEM((_T8, P_, 8), jnp.float32),
        ],
        compiler_params=pltpu.CompilerParams(
            dimension_semantics=("parallel",)),
    )

    for l in range(L_):
        hT = layer_call(rT, r, hT, featneT, nuc_embT,
                        WwT[l], WhT[l], WuT[l])
    return hT.T
